# Initial kernel scaffold; baseline (speedup 1.0000x reference)
import jax
import jax.numpy as jnp
from jax.experimental import pallas as pl


def kernel(X, Y):
    raise NotImplementedError("write your pallas kernel here")


# speedup vs baseline: 2.1811x; 2.1811x over previous
import functools

import jax
import jax.numpy as jnp
from jax import lax
from jax.experimental import pallas as pl
from jax.experimental.pallas import tpu as pltpu

N = 4096
D = 1024
P = 128
C = 2 * P

_SCHED = [(k, j) for k in range(1, 13) for j in range(k - 1, -1, -1)]
N_SUB = len(_SCHED)


def _projections():
    k = jax.random.key(0)
    proj = jax.random.normal(k, (D, P), dtype=jnp.float32)
    proj = proj / jnp.sqrt(jnp.sum(proj**2, axis=0, keepdims=True))
    return proj


def _substage(b_ref, kbit, d):
    m = N // (2 * d)
    rows = lax.broadcasted_iota(jnp.int32, (N, 1), 0)
    asc = ((rows >> kbit) & 1) == 0
    v = b_ref[...].reshape(m, 2 * d, C)
    a3 = asc.reshape(m, 2 * d, 1)
    asc_b = a3[:, 0:1, :]
    lo = v[:, :d, :]
    hi = v[:, d:, :]
    mn = jnp.minimum(lo, hi)
    mx = jnp.maximum(lo, hi)
    new_lo = jnp.where(asc_b, mn, mx)
    new_hi = jnp.where(asc_b, mx, mn)
    b_ref[...] = jnp.concatenate([new_lo, new_hi], axis=1).reshape(N, C)


def _body(jj_ref, kk_ref, x_ref, y_ref, p_ref, out_ref, b_ref):
    s = pl.program_id(0)

    @pl.when(s == 0)
    def _init():
        b_ref[:, :P] = jnp.dot(x_ref[...], p_ref[...],
                               preferred_element_type=jnp.float32)
        b_ref[:, P:] = jnp.dot(y_ref[...], p_ref[...],
                               preferred_element_type=jnp.float32)

    @pl.when(s > 0)
    def _sort():
        t = s - 1
        jv = jj_ref[t]
        kv = kk_ref[t]
        branches = [functools.partial(_substage, b_ref, kv, 1 << j)
                    for j in range(12)]
        lax.switch(jv, branches)

    @pl.when(s == N_SUB)
    def _reduce():
        diff = b_ref[:, :P] - b_ref[:, P:]
        cost = jnp.sum(diff * diff, keepdims=True).reshape(1, 1)
        out_ref[...] = jnp.sqrt(cost * (1.0 / (N * P)))


def kernel(X, Y):
    proj = _projections()
    jj = jnp.array([j for _, j in _SCHED], dtype=jnp.int32)
    kk = jnp.array([k for k, _ in _SCHED], dtype=jnp.int32)
    smem = pl.BlockSpec(memory_space=pltpu.SMEM)
    vmem = pl.BlockSpec(memory_space=pltpu.VMEM)
    out = pl.pallas_call(
        _body,
        grid=(N_SUB + 1,),
        in_specs=[smem, smem, vmem, vmem, vmem],
        out_specs=pl.BlockSpec(memory_space=pltpu.VMEM),
        out_shape=jax.ShapeDtypeStruct((1, 1), jnp.float32),
        scratch_shapes=[pltpu.VMEM((N, C), jnp.float32)],
    )(jj, kk, X, Y, proj)
    return out[0, 0]
